# Initial kernel scaffold; baseline (speedup 1.0000x reference)
import jax
import jax.numpy as jnp
from jax.experimental import pallas as pl


def kernel(x, POI):
    raise NotImplementedError("write your pallas kernel here")


# speedup vs baseline: 4.1938x; 4.1938x over previous
import functools

import jax
import jax.numpy as jnp
from jax import lax
from jax.experimental import pallas as pl
from jax.experimental.pallas import tpu as pltpu
from jax.experimental.pallas import tpu_sc as plsc

NUM_LOCS = 100000
POI_DIM = 32
BATCH = 4096
HIST_LEN = 200

NC = 2
NS = 16
NW = NC * NS

ROWS = BATCH * HIST_LEN
RPW = ROWS // NW
CH = 128
NCH = RPW // CH

_mesh = plsc.VectorSubcoreMesh(core_axis_name="c", subcore_axis_name="s")


@functools.partial(
    pl.kernel,
    mesh=_mesh,
    compiler_params=pltpu.CompilerParams(use_tc_tiling_on_sc=False),
    out_type=jax.ShapeDtypeStruct((NW, NCH, CH, POI_DIM), jnp.float32),
    scratch_types=[
        pltpu.VMEM((NCH, CH), jnp.int32),
        pltpu.VMEM((CH, POI_DIM), jnp.float32),
        pltpu.SemaphoreType.DMA,
    ],
)
def _poi_gather(x_hbm, tab_hbm, out_hbm, idx_v, rows_v, gsem):
    wid = lax.axis_index("s") * NC + lax.axis_index("c")
    pltpu.sync_copy(x_hbm.at[wid], idx_v)

    def body(j, carry):
        pltpu.async_copy(tab_hbm.at[idx_v.at[j]], rows_v, gsem).wait()
        pltpu.sync_copy(rows_v, out_hbm.at[wid, j])
        return carry

    lax.fori_loop(0, NCH, body, 0)


def kernel(x, POI):
    xi = x.reshape(NW, NCH, CH).astype(jnp.int32)
    out = _poi_gather(xi, POI)
    return out.reshape(BATCH, HIST_LEN, POI_DIM)


# speedup vs baseline: 5.2599x; 1.2542x over previous
import functools

import jax
import jax.numpy as jnp
from jax import lax
from jax.experimental import pallas as pl
from jax.experimental.pallas import tpu as pltpu
from jax.experimental.pallas import tpu_sc as plsc

NUM_LOCS = 100000
POI_DIM = 32
BATCH = 4096
HIST_LEN = 200

NC = 2
NS = 16
NW = NC * NS

ROWS = BATCH * HIST_LEN
RPW = ROWS // NW
CH = 128
NCH = RPW // CH
G = 10
NG = NCH // G
GR = G * CH

_mesh = plsc.VectorSubcoreMesh(core_axis_name="c", subcore_axis_name="s")


@functools.partial(
    pl.kernel,
    mesh=_mesh,
    compiler_params=pltpu.CompilerParams(use_tc_tiling_on_sc=False),
    out_type=jax.ShapeDtypeStruct((NW, NG, GR, POI_DIM), jnp.float32),
    scratch_types=[
        pltpu.VMEM((NCH, CH), jnp.int32),
        pltpu.VMEM((GR, POI_DIM), jnp.float32),
        pltpu.VMEM((GR, POI_DIM), jnp.float32),
        pltpu.SemaphoreType.DMA,
        pltpu.SemaphoreType.DMA,
    ],
)
def _poi_gather(x_hbm, tab_hbm, out_hbm, idx_v, rows0, rows1, gsem, osem):
    wid = lax.axis_index("s") * NC + lax.axis_index("c")
    pltpu.sync_copy(x_hbm.at[wid], idx_v)
    bufs = (rows0, rows1)

    def fire_gathers(g, buf):
        for k in range(G):
            pltpu.async_copy(
                tab_hbm.at[idx_v.at[g * G + k]],
                buf.at[pl.ds(k * CH, CH)],
                gsem,
            )

    def wait_gathers(buf):
        pltpu.make_async_copy(tab_hbm.at[pl.ds(0, GR)], buf, gsem).wait()

    def wait_out(buf):
        pltpu.make_async_copy(tab_hbm.at[pl.ds(0, GR)], buf, osem).wait()

    fire_gathers(0, bufs[0])

    def body(gg, carry):
        for phase in range(2):
            g = gg * 2 + phase
            buf = bufs[phase]
            other = bufs[1 - phase]
            wait_gathers(buf)
            pltpu.async_copy(buf, out_hbm.at[wid, g], osem)

            @pl.when(g >= 1)
            def _():
                wait_out(other)

            @pl.when(g + 1 < NG)
            def _():
                fire_gathers(g + 1, other)

        return carry

    lax.fori_loop(0, NG // 2, body, 0)
    wait_out(bufs[1])


def kernel(x, POI):
    xi = x.reshape(NW, NCH, CH).astype(jnp.int32)
    out = _poi_gather(xi, POI)
    return out.reshape(BATCH, HIST_LEN, POI_DIM)


# speedup vs baseline: 5.3142x; 1.0103x over previous
import functools

import jax
import jax.numpy as jnp
from jax import lax
from jax.experimental import pallas as pl
from jax.experimental.pallas import tpu as pltpu
from jax.experimental.pallas import tpu_sc as plsc

NUM_LOCS = 100000
POI_DIM = 32
BATCH = 4096
HIST_LEN = 200

NC = 2
NS = 16
NW = NC * NS

ROWS = BATCH * HIST_LEN
RPW = ROWS // NW
CH = 128
NCH = RPW // CH
G = 5
NG = NCH // G
GR = G * CH
NBUF = 4

_mesh = plsc.VectorSubcoreMesh(core_axis_name="c", subcore_axis_name="s")


@functools.partial(
    pl.kernel,
    mesh=_mesh,
    compiler_params=pltpu.CompilerParams(use_tc_tiling_on_sc=False),
    out_type=jax.ShapeDtypeStruct((NW, NG, GR, POI_DIM), jnp.float32),
    name="poi_gather",
    scratch_types=[
        pltpu.VMEM((NCH, CH), jnp.int32),
        pltpu.VMEM((NBUF, GR, POI_DIM), jnp.float32),
        pltpu.SemaphoreType.DMA,
        pltpu.SemaphoreType.DMA,
    ],
)
def _poi_gather(x_hbm, tab_hbm, out_hbm, idx_v, rows_v, gsem, osem):
    wid = lax.axis_index("s") * NC + lax.axis_index("c")
    pltpu.sync_copy(x_hbm.at[wid], idx_v)

    def fire_gathers(g, b):
        for k in range(G):
            pltpu.async_copy(
                tab_hbm.at[idx_v.at[g * G + k]],
                rows_v.at[b, pl.ds(k * CH, CH)],
                gsem,
            )

    def wait_gathers(b):
        pltpu.make_async_copy(tab_hbm.at[pl.ds(0, GR)], rows_v.at[b], gsem).wait()

    def wait_out(b):
        pltpu.make_async_copy(tab_hbm.at[pl.ds(0, GR)], rows_v.at[b], osem).wait()

    fire_gathers(0, 0)
    fire_gathers(1, 1)

    def body(gg, carry):
        for phase in range(NBUF):
            g = gg * NBUF + phase
            b = phase
            wait_gathers(b)
            pltpu.async_copy(rows_v.at[b], out_hbm.at[wid, g], osem)

            @pl.when(g >= 2)
            def _():
                wait_out((phase + 2) % NBUF)

            @pl.when(g + 2 < NG)
            def _():
                fire_gathers(g + 2, (phase + 2) % NBUF)

        return carry

    lax.fori_loop(0, NG // NBUF, body, 0)
    wait_out(NBUF - 2)
    wait_out(NBUF - 1)


def kernel(x, POI):
    xi = x.reshape(NW, NCH, CH).astype(jnp.int32)
    out = _poi_gather(xi, POI)
    return out.reshape(BATCH, HIST_LEN, POI_DIM)
